# baseline (speedup 1.0000x reference)
import functools

import jax
import jax.numpy as jnp
from jax import lax
from jax.experimental import pallas as pl
from jax.experimental.pallas import tpu as pltpu
from jax.experimental.pallas import tpu_sc as plsc

_LAT_MIN, _LAT_MAX = -90.0, 90.0
_LON_MIN, _LON_MAX = -180.0, 180.0

_NC, _NS, _LANES = 2, 16, 16
_LB = 8
_W = 256
_NBUF = 2


def _splat(coef_v, i):
    return plsc.load_gather(coef_v, [jnp.full((_LANES,), i, jnp.int32)])


def _sc_body(idx_hbm, loc_hbm, tab_hbm, coef_hbm, out_hbm,
             idx_v, loc_v, out_v, plane_v, coef_v, sem_in, sem_out):
    cid = lax.axis_index("c")
    tid = lax.axis_index("s")
    l_total = idx_hbm.shape[0]
    b_half = idx_hbm.shape[1] // _NC
    b_base = cid * b_half
    n_lb = l_total // _LB
    n_bc = b_half // _W
    n_chunks = n_lb * n_bc

    is_region = tid >= 6

    pltpu.sync_copy(coef_hbm, coef_v)
    av = _splat(coef_v, tid)
    cv = _splat(coef_v, 16 + tid)
    dv = _splat(coef_v, 32 + tid)

    @pl.when(is_region)
    def _stage_plane():
        pltpu.make_async_copy(
            tab_hbm.at[tid - 6], plane_v, sem_in.at[0]).start()
        pltpu.make_async_copy(
            tab_hbm.at[tid - 6], plane_v, sem_in.at[0]).wait()

    def _in_start(c, p):
        lb = c // n_bc
        bc = c - lb * n_bc
        l0 = pl.multiple_of(lb * _LB, _LB)
        b0 = pl.multiple_of(b_base + bc * _W, _W)

        @pl.when(is_region)
        def _():
            pltpu.make_async_copy(
                idx_hbm.at[pl.ds(l0, _LB), pl.ds(b0, _W)],
                idx_v.at[p], sem_in.at[p]).start()

        @pl.when(jnp.logical_not(is_region))
        def _():
            pltpu.make_async_copy(
                loc_hbm.at[pl.ds(l0, _LB), :, pl.ds(b0, _W)],
                loc_v.at[p], sem_in.at[p]).start()

    def _in_wait(p):
        @pl.when(is_region)
        def _():
            pltpu.make_async_copy(
                idx_hbm.at[pl.ds(0, _LB), pl.ds(0, _W)],
                idx_v.at[p], sem_in.at[p]).wait()

        @pl.when(jnp.logical_not(is_region))
        def _():
            pltpu.make_async_copy(
                loc_hbm.at[pl.ds(0, _LB), :, pl.ds(0, _W)],
                loc_v.at[p], sem_in.at[p]).wait()

    def _out_start(c, p):
        lb = c // n_bc
        bc = c - lb * n_bc
        l0 = pl.multiple_of(lb * _LB, _LB)
        b0 = pl.multiple_of(b_base + bc * _W, _W)
        pltpu.make_async_copy(
            out_v.at[p],
            out_hbm.at[pl.ds(l0, _LB), tid, pl.ds(b0, _W)],
            sem_out.at[p]).start()

    def _out_wait(p):
        pltpu.make_async_copy(
            out_v.at[p],
            out_hbm.at[pl.ds(0, _LB), 0, pl.ds(0, _W)],
            sem_out.at[p]).wait()

    for p in range(_NBUF):
        _in_start(p, p)

    def chunk_body(c, carry):
        p = c % _NBUF
        _in_wait(p)

        @pl.when(c >= _NBUF)
        def _():
            _out_wait(p)

        @pl.when(is_region)
        def _compute_region():
            def row(r, acc):
                for k in range(_W // _LANES):
                    ids = idx_v[p, r, pl.ds(k * _LANES, _LANES)]
                    out_v[p, r, pl.ds(k * _LANES, _LANES)] = (
                        plsc.load_gather(plane_v, [ids]))
                return acc
            lax.fori_loop(0, _LB, row, 0)

        @pl.when(jnp.logical_not(is_region))
        def _compute_coord():
            def row(r, acc):
                for k in range(_W // _LANES):
                    lat = loc_v[p, r, 0, pl.ds(k * _LANES, _LANES)]
                    lon = loc_v[p, r, 1, pl.ds(k * _LANES, _LANES)]
                    out_v[p, r, pl.ds(k * _LANES, _LANES)] = (
                        lat * av + lon * cv + dv)
                return acc
            lax.fori_loop(0, _LB, row, 0)

        _out_start(c, p)

        @pl.when(c + _NBUF < n_chunks)
        def _():
            _in_start(c + _NBUF, p)
        return carry

    lax.fori_loop(0, n_chunks, chunk_body, 0)

    for p in range(_NBUF):
        _out_wait(p)


def kernel(location, region_id, coord_W, coord_b, region_table):
    B, L, _ = location.shape
    R = region_table.shape[0]

    a = coord_W[:, 0] / (_LAT_MAX - _LAT_MIN)
    c = coord_W[:, 1] / (_LON_MAX - _LON_MIN)
    d = coord_b + 0.5 * coord_W[:, 0] + 0.5 * coord_W[:, 1]
    pad = jnp.zeros((10,), jnp.float32)
    coefs = jnp.concatenate([a, pad, c, pad, d, pad])

    idx_t = region_id.T
    loc_t = jnp.transpose(location, (1, 2, 0))
    tab_t = region_table.T

    mesh = plsc.VectorSubcoreMesh(core_axis_name="c", subcore_axis_name="s")
    run = functools.partial(
        pl.kernel,
        mesh=mesh,
        out_type=jax.ShapeDtypeStruct((L, 16, B), jnp.float32),
        scratch_types=[
            pltpu.VMEM((_NBUF, _LB, _W), jnp.int32),
            pltpu.VMEM((_NBUF, _LB, 2, _W), jnp.float32),
            pltpu.VMEM((_NBUF, _LB, _W), jnp.float32),
            pltpu.VMEM((R,), jnp.float32),
            pltpu.VMEM((48,), jnp.float32),
            pltpu.SemaphoreType.DMA((_NBUF,)),
            pltpu.SemaphoreType.DMA((_NBUF,)),
        ],
        compiler_params=pltpu.CompilerParams(
            needs_layout_passes=False, use_tc_tiling_on_sc=True),
    )(_sc_body)
    out_t = run(idx_t, loc_t, tab_t, coefs)
    return jnp.transpose(out_t, (2, 0, 1))


# speedup vs baseline: 1.9518x; 1.9518x over previous
import functools

import jax
import jax.numpy as jnp
from jax import lax
from jax.experimental import pallas as pl
from jax.experimental.pallas import tpu as pltpu
from jax.experimental.pallas import tpu_sc as plsc

_LAT_MIN, _LAT_MAX = -90.0, 90.0
_LON_MIN, _LON_MAX = -180.0, 180.0

_NC, _NS, _LANES = 2, 16, 16
_LB = 8
_W = 256
_NBUF = 2


def _splat(coef_v, i):
    return plsc.load_gather(coef_v, [jnp.full((_LANES,), i, jnp.int32)])


def _sc_body(idx_hbm, loc_hbm, tab_hbm, coef_hbm, out_hbm,
             idx_v, loc_v, out_v, plane_v, coef_v, sem_in, sem_out):
    cid = lax.axis_index("c")
    tid = lax.axis_index("s")
    l_total = idx_hbm.shape[0]
    b_half = idx_hbm.shape[1] // _NC
    b_base = cid * b_half
    n_lb = l_total // _LB
    n_bc = b_half // _W
    n_chunks = n_lb * n_bc

    is_region = tid >= 6

    pltpu.sync_copy(coef_hbm, coef_v)
    av = _splat(coef_v, tid)
    cv = _splat(coef_v, 16 + tid)
    dv = _splat(coef_v, 32 + tid)

    @pl.when(is_region)
    def _stage_plane():
        pltpu.make_async_copy(
            tab_hbm.at[tid - 6], plane_v, sem_in.at[0]).start()
        pltpu.make_async_copy(
            tab_hbm.at[tid - 6], plane_v, sem_in.at[0]).wait()

    def _in_start(c, p):
        lb = c // n_bc
        bc = c - lb * n_bc
        l0 = pl.multiple_of(lb * _LB, _LB)
        b0 = pl.multiple_of(b_base + bc * _W, _W)

        @pl.when(is_region)
        def _():
            pltpu.make_async_copy(
                idx_hbm.at[pl.ds(l0, _LB), pl.ds(b0, _W)],
                idx_v.at[p], sem_in.at[p]).start()

        @pl.when(jnp.logical_not(is_region))
        def _():
            pltpu.make_async_copy(
                loc_hbm.at[pl.ds(l0, _LB), :, pl.ds(b0, _W)],
                loc_v.at[p], sem_in.at[p]).start()

    def _in_wait(p):
        @pl.when(is_region)
        def _():
            pltpu.make_async_copy(
                idx_hbm.at[pl.ds(0, _LB), pl.ds(0, _W)],
                idx_v.at[p], sem_in.at[p]).wait()

        @pl.when(jnp.logical_not(is_region))
        def _():
            pltpu.make_async_copy(
                loc_hbm.at[pl.ds(0, _LB), :, pl.ds(0, _W)],
                loc_v.at[p], sem_in.at[p]).wait()

    def _out_start(c, p):
        lb = c // n_bc
        bc = c - lb * n_bc
        l0 = pl.multiple_of(lb * _LB, _LB)
        b0 = pl.multiple_of(b_base + bc * _W, _W)
        pltpu.make_async_copy(
            out_v.at[p],
            out_hbm.at[pl.ds(l0, _LB), tid, pl.ds(b0, _W)],
            sem_out.at[p]).start()

    def _out_wait(p):
        pltpu.make_async_copy(
            out_v.at[p],
            out_hbm.at[pl.ds(0, _LB), 0, pl.ds(0, _W)],
            sem_out.at[p]).wait()

    for p in range(_NBUF):
        _in_start(p, p)

    def chunk_body(c, carry):
        p = c % _NBUF
        _in_wait(p)

        @pl.when(c >= _NBUF)
        def _():
            _out_wait(p)

        n_vec = _W // _LANES

        @pl.when(is_region)
        def _compute_region():
            @plsc.parallel_loop(0, _LB * n_vec, unroll=4)
            def _(i):
                r = lax.shift_right_logical(i, 4)
                k = lax.bitwise_and(i, n_vec - 1)
                ids = idx_v[p, r, pl.ds(k * _LANES, _LANES)]
                out_v[p, r, pl.ds(k * _LANES, _LANES)] = (
                    plsc.load_gather(plane_v, [ids]))

        @pl.when(jnp.logical_not(is_region))
        def _compute_coord():
            @plsc.parallel_loop(0, _LB * n_vec, unroll=4)
            def _(i):
                r = lax.shift_right_logical(i, 4)
                k = lax.bitwise_and(i, n_vec - 1)
                lat = loc_v[p, r, 0, pl.ds(k * _LANES, _LANES)]
                lon = loc_v[p, r, 1, pl.ds(k * _LANES, _LANES)]
                out_v[p, r, pl.ds(k * _LANES, _LANES)] = (
                    lat * av + lon * cv + dv)

        _out_start(c, p)

        @pl.when(c + _NBUF < n_chunks)
        def _():
            _in_start(c + _NBUF, p)
        return carry

    lax.fori_loop(0, n_chunks, chunk_body, 0)

    for p in range(_NBUF):
        _out_wait(p)


def kernel(location, region_id, coord_W, coord_b, region_table):
    B, L, _ = location.shape
    R = region_table.shape[0]

    a = coord_W[:, 0] / (_LAT_MAX - _LAT_MIN)
    c = coord_W[:, 1] / (_LON_MAX - _LON_MIN)
    d = coord_b + 0.5 * coord_W[:, 0] + 0.5 * coord_W[:, 1]
    pad = jnp.zeros((10,), jnp.float32)
    coefs = jnp.concatenate([a, pad, c, pad, d, pad])

    idx_t = region_id.T
    loc_t = jnp.transpose(location, (1, 2, 0))
    tab_t = region_table.T

    mesh = plsc.VectorSubcoreMesh(core_axis_name="c", subcore_axis_name="s")
    run = functools.partial(
        pl.kernel,
        mesh=mesh,
        out_type=jax.ShapeDtypeStruct((L, 16, B), jnp.float32),
        scratch_types=[
            pltpu.VMEM((_NBUF, _LB, _W), jnp.int32),
            pltpu.VMEM((_NBUF, _LB, 2, _W), jnp.float32),
            pltpu.VMEM((_NBUF, _LB, _W), jnp.float32),
            pltpu.VMEM((R,), jnp.float32),
            pltpu.VMEM((48,), jnp.float32),
            pltpu.SemaphoreType.DMA((_NBUF,)),
            pltpu.SemaphoreType.DMA((_NBUF,)),
        ],
        compiler_params=pltpu.CompilerParams(
            needs_layout_passes=False, use_tc_tiling_on_sc=True),
    )(_sc_body)
    out_t = run(idx_t, loc_t, tab_t, coefs)
    return jnp.transpose(out_t, (2, 0, 1))


# speedup vs baseline: 2.6888x; 1.3776x over previous
import functools

import jax
import jax.numpy as jnp
from jax import lax
from jax.experimental import pallas as pl
from jax.experimental.pallas import tpu as pltpu
from jax.experimental.pallas import tpu_sc as plsc

_LAT_MIN, _LAT_MAX = -90.0, 90.0
_LON_MIN, _LON_MAX = -180.0, 180.0

_NC, _NS, _LANES = 2, 16, 16
_LB = 8
_WR = 512
_WC = 512
_NBUF = 2


def _splat(coef_v, i):
    return plsc.load_gather(coef_v, [jnp.full((_LANES,), i, jnp.int32)])


def _pipeline(n_chunks, in_start, in_wait, compute, out_start, out_wait):
    for p in range(_NBUF):
        in_start(p, p)

    def chunk_body(c, carry):
        p = c % _NBUF
        in_wait(p)

        @pl.when(c >= _NBUF)
        def _():
            out_wait(p)

        compute(p)
        out_start(c, p)

        @pl.when(c + _NBUF < n_chunks)
        def _():
            in_start(c + _NBUF, p)
        return carry

    lax.fori_loop(0, n_chunks, chunk_body, 0)
    for p in range(_NBUF):
        out_wait(p)


def _sc_body(idx_hbm, loc_hbm, tab_hbm, coef_hbm, out_hbm,
             coef_v, sem_in, sem_out):
    cid = lax.axis_index("c")
    tid = lax.axis_index("s")
    l_total = idx_hbm.shape[0]
    b_half = idx_hbm.shape[1] // _NC
    b_base = cid * b_half
    n_lb = l_total // _LB

    pltpu.sync_copy(coef_hbm, coef_v)
    av = _splat(coef_v, tid)
    cv = _splat(coef_v, 16 + tid)
    dv = _splat(coef_v, 32 + tid)

    def _coords(c, n_bc, w):
        lb = c // n_bc
        bc = c - lb * n_bc
        l0 = pl.multiple_of(lb * _LB, _LB)
        b0 = pl.multiple_of(b_base + bc * w, w)
        return l0, b0

    @pl.when(tid >= 6)
    def _region_role():
        def scoped(plane_v, idx_v, out_v):
            n_bc = b_half // _WR
            n_chunks = n_lb * n_bc
            n_vec = _WR // _LANES

            pltpu.make_async_copy(
                tab_hbm.at[tid - 6], plane_v, sem_in.at[0]).start()
            pltpu.make_async_copy(
                tab_hbm.at[tid - 6], plane_v, sem_in.at[0]).wait()

            def in_start(c, p):
                l0, b0 = _coords(c, n_bc, _WR)
                pltpu.make_async_copy(
                    idx_hbm.at[pl.ds(l0, _LB), pl.ds(b0, _WR)],
                    idx_v.at[p], sem_in.at[p]).start()

            def in_wait(p):
                pltpu.make_async_copy(
                    idx_hbm.at[pl.ds(0, _LB), pl.ds(0, _WR)],
                    idx_v.at[p], sem_in.at[p]).wait()

            def compute(p):
                @plsc.parallel_loop(0, _LB * n_vec, unroll=4)
                def _(i):
                    r = lax.shift_right_logical(i, 5)
                    k = lax.bitwise_and(i, n_vec - 1)
                    ids = idx_v[p, r, pl.ds(k * _LANES, _LANES)]
                    out_v[p, r, pl.ds(k * _LANES, _LANES)] = (
                        plsc.load_gather(plane_v, [ids]))

            def out_start(c, p):
                l0, b0 = _coords(c, n_bc, _WR)
                pltpu.make_async_copy(
                    out_v.at[p],
                    out_hbm.at[pl.ds(l0, _LB), tid, pl.ds(b0, _WR)],
                    sem_out.at[p]).start()

            def out_wait(p):
                pltpu.make_async_copy(
                    out_v.at[p],
                    out_hbm.at[pl.ds(0, _LB), 0, pl.ds(0, _WR)],
                    sem_out.at[p]).wait()

            _pipeline(n_chunks, in_start, in_wait, compute,
                      out_start, out_wait)

        pl.run_scoped(
            scoped,
            pltpu.VMEM((tab_hbm.shape[1],), jnp.float32),
            pltpu.VMEM((_NBUF, _LB, _WR), jnp.int32),
            pltpu.VMEM((_NBUF, _LB, _WR), jnp.float32),
        )

    @pl.when(tid < 6)
    def _coord_role():
        def scoped(loc_v, out_v):
            n_bc = b_half // _WC
            n_chunks = n_lb * n_bc
            n_vec = _WC // _LANES

            def in_start(c, p):
                l0, b0 = _coords(c, n_bc, _WC)
                pltpu.make_async_copy(
                    loc_hbm.at[pl.ds(l0, _LB), :, pl.ds(b0, _WC)],
                    loc_v.at[p], sem_in.at[p]).start()

            def in_wait(p):
                pltpu.make_async_copy(
                    loc_hbm.at[pl.ds(0, _LB), :, pl.ds(0, _WC)],
                    loc_v.at[p], sem_in.at[p]).wait()

            def compute(p):
                @plsc.parallel_loop(0, _LB * n_vec, unroll=4)
                def _(i):
                    r = lax.shift_right_logical(i, 5)
                    k = lax.bitwise_and(i, n_vec - 1)
                    lat = loc_v[p, r, 0, pl.ds(k * _LANES, _LANES)]
                    lon = loc_v[p, r, 1, pl.ds(k * _LANES, _LANES)]
                    out_v[p, r, pl.ds(k * _LANES, _LANES)] = (
                        lat * av + lon * cv + dv)

            def out_start(c, p):
                l0, b0 = _coords(c, n_bc, _WC)
                pltpu.make_async_copy(
                    out_v.at[p],
                    out_hbm.at[pl.ds(l0, _LB), tid, pl.ds(b0, _WC)],
                    sem_out.at[p]).start()

            def out_wait(p):
                pltpu.make_async_copy(
                    out_v.at[p],
                    out_hbm.at[pl.ds(0, _LB), 0, pl.ds(0, _WC)],
                    sem_out.at[p]).wait()

            _pipeline(n_chunks, in_start, in_wait, compute,
                      out_start, out_wait)

        pl.run_scoped(
            scoped,
            pltpu.VMEM((_NBUF, _LB, 2, _WC), jnp.float32),
            pltpu.VMEM((_NBUF, _LB, _WC), jnp.float32),
        )


def kernel(location, region_id, coord_W, coord_b, region_table):
    B, L, _ = location.shape
    R = region_table.shape[0]

    a = coord_W[:, 0] / (_LAT_MAX - _LAT_MIN)
    c = coord_W[:, 1] / (_LON_MAX - _LON_MIN)
    d = coord_b + 0.5 * coord_W[:, 0] + 0.5 * coord_W[:, 1]
    pad = jnp.zeros((10,), jnp.float32)
    coefs = jnp.concatenate([a, pad, c, pad, d, pad])

    idx_t = region_id.T
    loc_t = jnp.transpose(location, (1, 2, 0))
    tab_t = region_table.T

    mesh = plsc.VectorSubcoreMesh(core_axis_name="c", subcore_axis_name="s")
    run = functools.partial(
        pl.kernel,
        mesh=mesh,
        out_type=jax.ShapeDtypeStruct((L, 16, B), jnp.float32),
        scratch_types=[
            pltpu.VMEM((48,), jnp.float32),
            pltpu.SemaphoreType.DMA((_NBUF,)),
            pltpu.SemaphoreType.DMA((_NBUF,)),
        ],
        compiler_params=pltpu.CompilerParams(
            needs_layout_passes=False, use_tc_tiling_on_sc=True),
    )(_sc_body)
    out_t = run(idx_t, loc_t, tab_t, coefs)
    return jnp.transpose(out_t, (2, 0, 1))


# speedup vs baseline: 3.0779x; 1.1447x over previous
import functools

import jax
import jax.numpy as jnp
from jax import lax
from jax.experimental import pallas as pl
from jax.experimental.pallas import tpu as pltpu
from jax.experimental.pallas import tpu_sc as plsc

_LAT_MIN, _LAT_MAX = -90.0, 90.0
_LON_MIN, _LON_MAX = -180.0, 180.0

_NC, _NS, _LANES = 2, 16, 16
_LB = 8
_WR = 512
_WC = 512
_NBUF = 2


def _splat(coef_v, i):
    return plsc.load_gather(coef_v, [jnp.full((_LANES,), i, jnp.int32)])


def _pipeline(n_chunks, in_start, in_wait, compute, out_start, out_wait):
    for p in range(_NBUF):
        in_start(p, p)

    def chunk_body(c, carry):
        p = c % _NBUF
        in_wait(p)

        @pl.when(c >= _NBUF)
        def _():
            out_wait(p)

        compute(p)
        out_start(c, p)

        @pl.when(c + _NBUF < n_chunks)
        def _():
            in_start(c + _NBUF, p)
        return carry

    lax.fori_loop(0, n_chunks, chunk_body, 0)
    for p in range(_NBUF):
        out_wait(p)


def _sc_body(idx_hbm, loc_hbm, tab_hbm, coef_hbm, out_hbm,
             coef_v, sem_in, sem_out):
    cid = lax.axis_index("c")
    tid = lax.axis_index("s")
    l_total = idx_hbm.shape[0]
    b_half = idx_hbm.shape[1] // _NC
    b_base = cid * b_half
    n_lb = l_total // _LB

    pltpu.sync_copy(coef_hbm, coef_v)
    av = _splat(coef_v, tid)
    cv = _splat(coef_v, 16 + tid)
    dv = _splat(coef_v, 32 + tid)

    def _coords(c, n_bc, w):
        lb = c // n_bc
        bc = c - lb * n_bc
        l0 = pl.multiple_of(lb * _LB, _LB)
        b0 = pl.multiple_of(b_base + bc * w, w)
        return l0, b0

    @pl.when(tid >= 6)
    def _region_role():
        def scoped(plane_v, idx_v, out_v):
            n_bc = b_half // _WR
            n_chunks = n_lb * n_bc
            n_vec = _WR // _LANES

            pltpu.make_async_copy(
                tab_hbm.at[tid - 6], plane_v, sem_in.at[0]).start()
            pltpu.make_async_copy(
                tab_hbm.at[tid - 6], plane_v, sem_in.at[0]).wait()

            def in_start(c, p):
                l0, b0 = _coords(c, n_bc, _WR)
                pltpu.make_async_copy(
                    idx_hbm.at[pl.ds(l0, _LB), pl.ds(b0, _WR)],
                    idx_v.at[p], sem_in.at[p]).start()

            def in_wait(p):
                pltpu.make_async_copy(
                    idx_hbm.at[pl.ds(0, _LB), pl.ds(0, _WR)],
                    idx_v.at[p], sem_in.at[p]).wait()

            def compute(p):
                @plsc.parallel_loop(0, _LB * n_vec, unroll=4)
                def _(i):
                    r = lax.shift_right_logical(i, 5)
                    k = lax.bitwise_and(i, n_vec - 1)
                    ids = idx_v[p, r, pl.ds(k * _LANES, _LANES)]
                    out_v[p, r, pl.ds(k * _LANES, _LANES)] = (
                        plsc.load_gather(plane_v, [ids]))

            def out_start(c, p):
                l0, b0 = _coords(c, n_bc, _WR)
                pltpu.make_async_copy(
                    out_v.at[p],
                    out_hbm.at[pl.ds(l0, _LB), tid, pl.ds(b0, _WR)],
                    sem_out.at[p]).start()

            def out_wait(p):
                pltpu.make_async_copy(
                    out_v.at[p],
                    out_hbm.at[pl.ds(0, _LB), 0, pl.ds(0, _WR)],
                    sem_out.at[p]).wait()

            _pipeline(n_chunks, in_start, in_wait, compute,
                      out_start, out_wait)

        pl.run_scoped(
            scoped,
            pltpu.VMEM((tab_hbm.shape[1],), jnp.float32),
            pltpu.VMEM((_NBUF, _LB, _WR), jnp.int32),
            pltpu.VMEM((_NBUF, _LB, _WR), jnp.float32),
        )

    @pl.when(tid < 6)
    def _coord_role():
        j0 = (tid % 3) * 2
        b_q = b_half // 2
        qb_base = b_base + (tid // 3) * b_q
        avs = [_splat(coef_v, j0), _splat(coef_v, j0 + 1)]
        cvs = [_splat(coef_v, 16 + j0), _splat(coef_v, 16 + j0 + 1)]
        dvs = [_splat(coef_v, 32 + j0), _splat(coef_v, 32 + j0 + 1)]

        def scoped(loc_v, out_v):
            n_bc = b_q // _WC
            n_chunks = n_lb * n_bc
            n_vec = _WC // _LANES

            def qcoords(c):
                lb = c // n_bc
                bc = c - lb * n_bc
                l0 = pl.multiple_of(lb * _LB, _LB)
                b0 = pl.multiple_of(qb_base + bc * _WC, _WC)
                return l0, b0

            def in_start(c, p):
                l0, b0 = qcoords(c)
                pltpu.make_async_copy(
                    loc_hbm.at[pl.ds(l0, _LB), :, pl.ds(b0, _WC)],
                    loc_v.at[p], sem_in.at[p]).start()

            def in_wait(p):
                pltpu.make_async_copy(
                    loc_hbm.at[pl.ds(0, _LB), :, pl.ds(0, _WC)],
                    loc_v.at[p], sem_in.at[p]).wait()

            def compute(p):
                @plsc.parallel_loop(0, _LB * n_vec, unroll=4)
                def _(i):
                    r = lax.shift_right_logical(i, 5)
                    k = lax.bitwise_and(i, n_vec - 1)
                    lat = loc_v[p, r, 0, pl.ds(k * _LANES, _LANES)]
                    lon = loc_v[p, r, 1, pl.ds(k * _LANES, _LANES)]
                    for jj in range(2):
                        out_v[p, jj, r, pl.ds(k * _LANES, _LANES)] = (
                            lat * avs[jj] + lon * cvs[jj] + dvs[jj])

            def out_start(c, p):
                l0, b0 = qcoords(c)
                for jj in range(2):
                    pltpu.make_async_copy(
                        out_v.at[p, jj],
                        out_hbm.at[pl.ds(l0, _LB), j0 + jj, pl.ds(b0, _WC)],
                        sem_out.at[p]).start()

            def out_wait(p):
                for jj in range(2):
                    pltpu.make_async_copy(
                        out_v.at[p, jj],
                        out_hbm.at[pl.ds(0, _LB), 0, pl.ds(0, _WC)],
                        sem_out.at[p]).wait()

            _pipeline(n_chunks, in_start, in_wait, compute,
                      out_start, out_wait)

        pl.run_scoped(
            scoped,
            pltpu.VMEM((_NBUF, _LB, 2, _WC), jnp.float32),
            pltpu.VMEM((_NBUF, 2, _LB, _WC), jnp.float32),
        )


def kernel(location, region_id, coord_W, coord_b, region_table):
    B, L, _ = location.shape
    R = region_table.shape[0]

    a = coord_W[:, 0] / (_LAT_MAX - _LAT_MIN)
    c = coord_W[:, 1] / (_LON_MAX - _LON_MIN)
    d = coord_b + 0.5 * coord_W[:, 0] + 0.5 * coord_W[:, 1]
    pad = jnp.zeros((10,), jnp.float32)
    coefs = jnp.concatenate([a, pad, c, pad, d, pad])

    idx_t = region_id.T
    loc_t = jnp.transpose(location, (1, 2, 0))
    tab_t = region_table.T

    mesh = plsc.VectorSubcoreMesh(core_axis_name="c", subcore_axis_name="s")
    run = functools.partial(
        pl.kernel,
        mesh=mesh,
        out_type=jax.ShapeDtypeStruct((L, 16, B), jnp.float32),
        scratch_types=[
            pltpu.VMEM((48,), jnp.float32),
            pltpu.SemaphoreType.DMA((_NBUF,)),
            pltpu.SemaphoreType.DMA((_NBUF,)),
        ],
        compiler_params=pltpu.CompilerParams(
            needs_layout_passes=False, use_tc_tiling_on_sc=True),
    )(_sc_body)
    out_t = run(idx_t, loc_t, tab_t, coefs)
    return jnp.transpose(out_t, (2, 0, 1))


# speedup vs baseline: 4.0343x; 1.3107x over previous
import functools

import jax
import jax.numpy as jnp
from jax import lax
from jax.experimental import pallas as pl
from jax.experimental.pallas import tpu as pltpu
from jax.experimental.pallas import tpu_sc as plsc

_LAT_MIN, _LAT_MAX = -90.0, 90.0
_LON_MIN, _LON_MAX = -180.0, 180.0

_NC, _NS, _LANES = 2, 16, 16
_LB = 8
_WR = 512
_WC = 512
_NBUF = 3


def _splat(coef_v, i):
    return plsc.load_gather(coef_v, [jnp.full((_LANES,), i, jnp.int32)])


def _pipeline(n_chunks, in_start, in_wait, compute, out_start, out_wait):
    for p in range(_NBUF):
        in_start(p, p)

    def chunk_body(c, carry):
        p = c % _NBUF
        in_wait(p)

        @pl.when(c >= _NBUF)
        def _():
            out_wait(p)

        compute(p)
        out_start(c, p)

        @pl.when(c + _NBUF < n_chunks)
        def _():
            in_start(c + _NBUF, p)
        return carry

    lax.fori_loop(0, n_chunks, chunk_body, 0)
    for p in range(_NBUF):
        out_wait(p)


def _sc_body(idx_hbm, loc_hbm, tab_hbm, coef_hbm, out_hbm,
             coef_v, sem_in, sem_out):
    cid = lax.axis_index("c")
    tid = lax.axis_index("s")
    l_total = idx_hbm.shape[0]
    b_half = idx_hbm.shape[1] // _NC
    b_base = cid * b_half
    n_lb = l_total // _LB

    pltpu.sync_copy(coef_hbm, coef_v)
    av = _splat(coef_v, tid)
    cv = _splat(coef_v, 16 + tid)
    dv = _splat(coef_v, 32 + tid)

    def _coords(c, n_bc, w):
        lb = c // n_bc
        bc = c - lb * n_bc
        l0 = pl.multiple_of(lb * _LB, _LB)
        b0 = pl.multiple_of(b_base + bc * w, w)
        return l0, b0

    @pl.when(tid >= 6)
    def _region_role():
        def scoped(plane_v, idx_v, out_v):
            n_bc = b_half // _WR
            n_chunks = n_lb * n_bc
            n_vec = _WR // _LANES

            pltpu.make_async_copy(
                tab_hbm.at[tid - 6], plane_v, sem_in.at[0]).start()
            pltpu.make_async_copy(
                tab_hbm.at[tid - 6], plane_v, sem_in.at[0]).wait()

            def in_start(c, p):
                l0, b0 = _coords(c, n_bc, _WR)
                pltpu.make_async_copy(
                    idx_hbm.at[pl.ds(l0, _LB), pl.ds(b0, _WR)],
                    idx_v.at[p], sem_in.at[p]).start()

            def in_wait(p):
                pltpu.make_async_copy(
                    idx_hbm.at[pl.ds(0, _LB), pl.ds(0, _WR)],
                    idx_v.at[p], sem_in.at[p]).wait()

            def compute(p):
                @plsc.parallel_loop(0, _LB * n_vec, unroll=8)
                def _(i):
                    r = lax.shift_right_logical(i, 5)
                    k = lax.bitwise_and(i, n_vec - 1)
                    ids = idx_v[p, r, pl.ds(k * _LANES, _LANES)]
                    out_v[p, r, pl.ds(k * _LANES, _LANES)] = (
                        plsc.load_gather(plane_v, [ids]))

            def out_start(c, p):
                l0, b0 = _coords(c, n_bc, _WR)
                pltpu.make_async_copy(
                    out_v.at[p],
                    out_hbm.at[pl.ds(l0, _LB), tid, pl.ds(b0, _WR)],
                    sem_out.at[p]).start()

            def out_wait(p):
                pltpu.make_async_copy(
                    out_v.at[p],
                    out_hbm.at[pl.ds(0, _LB), 0, pl.ds(0, _WR)],
                    sem_out.at[p]).wait()

            _pipeline(n_chunks, in_start, in_wait, compute,
                      out_start, out_wait)

        pl.run_scoped(
            scoped,
            pltpu.VMEM((tab_hbm.shape[1],), jnp.float32),
            pltpu.VMEM((_NBUF, _LB, _WR), jnp.int32),
            pltpu.VMEM((_NBUF, _LB, _WR), jnp.float32),
        )

    @pl.when(tid < 6)
    def _coord_role():
        j0 = (tid % 3) * 2
        b_q = b_half // 2
        qb_base = b_base + (tid // 3) * b_q
        avs = [_splat(coef_v, j0), _splat(coef_v, j0 + 1)]
        cvs = [_splat(coef_v, 16 + j0), _splat(coef_v, 16 + j0 + 1)]
        dvs = [_splat(coef_v, 32 + j0), _splat(coef_v, 32 + j0 + 1)]

        def scoped(loc_v, out_v):
            n_bc = b_q // _WC
            n_chunks = n_lb * n_bc
            n_vec = _WC // _LANES

            def qcoords(c):
                lb = c // n_bc
                bc = c - lb * n_bc
                l0 = pl.multiple_of(lb * _LB, _LB)
                b0 = pl.multiple_of(qb_base + bc * _WC, _WC)
                return l0, b0

            def in_start(c, p):
                l0, b0 = qcoords(c)
                pltpu.make_async_copy(
                    loc_hbm.at[pl.ds(l0, _LB), :, pl.ds(b0, _WC)],
                    loc_v.at[p], sem_in.at[p]).start()

            def in_wait(p):
                pltpu.make_async_copy(
                    loc_hbm.at[pl.ds(0, _LB), :, pl.ds(0, _WC)],
                    loc_v.at[p], sem_in.at[p]).wait()

            def compute(p):
                @plsc.parallel_loop(0, _LB * n_vec, unroll=4)
                def _(i):
                    r = lax.shift_right_logical(i, 5)
                    k = lax.bitwise_and(i, n_vec - 1)
                    lat = loc_v[p, r, 0, pl.ds(k * _LANES, _LANES)]
                    lon = loc_v[p, r, 1, pl.ds(k * _LANES, _LANES)]
                    for jj in range(2):
                        out_v[p, jj, r, pl.ds(k * _LANES, _LANES)] = (
                            lat * avs[jj] + lon * cvs[jj] + dvs[jj])

            def out_start(c, p):
                l0, b0 = qcoords(c)
                for jj in range(2):
                    pltpu.make_async_copy(
                        out_v.at[p, jj],
                        out_hbm.at[pl.ds(l0, _LB), j0 + jj, pl.ds(b0, _WC)],
                        sem_out.at[p]).start()

            def out_wait(p):
                for jj in range(2):
                    pltpu.make_async_copy(
                        out_v.at[p, jj],
                        out_hbm.at[pl.ds(0, _LB), 0, pl.ds(0, _WC)],
                        sem_out.at[p]).wait()

            _pipeline(n_chunks, in_start, in_wait, compute,
                      out_start, out_wait)

        pl.run_scoped(
            scoped,
            pltpu.VMEM((_NBUF, _LB, 2, _WC), jnp.float32),
            pltpu.VMEM((_NBUF, 2, _LB, _WC), jnp.float32),
        )


def kernel(location, region_id, coord_W, coord_b, region_table):
    B, L, _ = location.shape
    R = region_table.shape[0]

    a = coord_W[:, 0] / (_LAT_MAX - _LAT_MIN)
    c = coord_W[:, 1] / (_LON_MAX - _LON_MIN)
    d = coord_b + 0.5 * coord_W[:, 0] + 0.5 * coord_W[:, 1]
    pad = jnp.zeros((10,), jnp.float32)
    coefs = jnp.concatenate([a, pad, c, pad, d, pad])

    idx_t = region_id.T
    loc_t = jnp.transpose(location, (1, 2, 0))
    tab_t = region_table.T

    mesh = plsc.VectorSubcoreMesh(core_axis_name="c", subcore_axis_name="s")
    run = functools.partial(
        pl.kernel,
        mesh=mesh,
        out_type=jax.ShapeDtypeStruct((L, 16, B), jnp.float32),
        scratch_types=[
            pltpu.VMEM((48,), jnp.float32),
            pltpu.SemaphoreType.DMA((_NBUF,)),
            pltpu.SemaphoreType.DMA((_NBUF,)),
        ],
        compiler_params=pltpu.CompilerParams(
            needs_layout_passes=False, use_tc_tiling_on_sc=True),
    )(_sc_body)
    out_t = run(idx_t, loc_t, tab_t, coefs)
    return jnp.transpose(out_t, (2, 0, 1))
